# baseline (speedup 1.0000x reference)
import functools
import jax
import jax.numpy as jnp
from jax import lax
from jax.experimental import pallas as pl
from jax.experimental.pallas import tpu as pltpu
from jax.experimental.pallas import tpu_sc as plsc

H = 512
W = 512
C = 17
P = H * W
NW = 32
RPW = H // NW
CH = 256
L = 16


def _sc_body(hm_hbm, off_hbm, out_hbm, hm_v, off_v, out_v):
    wid = lax.axis_index("s") * 2 + lax.axis_index("c")
    iota = lax.iota(jnp.int32, L)

    for r in range(RPW):
        row = wid * RPW + r
        for half in range(2):
            w0 = half * CH
            base_p = row * W + w0
            pltpu.sync_copy(hm_hbm.at[0, row, pl.ds(w0, CH)], hm_v)
            pltpu.sync_copy(off_hbm.at[0, row, pl.ds(w0, CH)], off_v)

            def body(j, _):
                p_loc = j * L + iota
                c0 = jnp.zeros((L,), jnp.int32)
                best = plsc.load_gather(hm_v, [p_loc, c0])
                bestc = c0
                for c in range(1, C):
                    v = plsc.load_gather(hm_v, [p_loc, c0 + c])
                    gt = v > best
                    best = jnp.where(gt, v, best)
                    bestc = jnp.where(gt, jnp.full((L,), c, jnp.int32), bestc)
                score = 1.0 / (1.0 + jnp.exp(-best))

                y_off = plsc.load_gather(off_v, [p_loc, bestc])
                x_off = plsc.load_gather(off_v, [p_loc, bestc + C])

                px = (w0 + p_loc).astype(jnp.float32)
                py = jnp.full((L,), 4.0, jnp.float32) * row
                xv = (px * 4.0 + x_off).astype(jnp.int32).astype(jnp.float32)
                yv = (py + y_off).astype(jnp.int32).astype(jnp.float32)

                plsc.store_scatter(out_v, [p_loc, c0], bestc.astype(jnp.float32))
                plsc.store_scatter(out_v, [p_loc, c0 + 1], score)
                plsc.store_scatter(out_v, [p_loc, c0 + 2], xv)
                plsc.store_scatter(out_v, [p_loc, c0 + 3], yv)
                return 0

            lax.fori_loop(0, CH // L, body, 0)
            pltpu.sync_copy(out_v, out_hbm.at[0, pl.ds(base_p, CH)])


def kernel(heatmaps_input, offsets_input):
    k = functools.partial(
        pl.kernel,
        out_type=jax.ShapeDtypeStruct((1, P, 4), jnp.float32),
        scratch_types=[
            pltpu.VMEM((CH, C), jnp.float32),
            pltpu.VMEM((CH, 2 * C), jnp.float32),
            pltpu.VMEM((CH, 4), jnp.float32),
        ],
        mesh=plsc.VectorSubcoreMesh(core_axis_name="c", subcore_axis_name="s"),
        compiler_params=pltpu.CompilerParams(
            use_tc_tiling_on_sc=True, needs_layout_passes=False
        ),
    )(_sc_body)
    return k(heatmaps_input, offsets_input)


# speedup vs baseline: 1.2875x; 1.2875x over previous
import functools
import jax
import jax.numpy as jnp
from jax import lax
from jax.experimental import pallas as pl
from jax.experimental.pallas import tpu as pltpu
from jax.experimental.pallas import tpu_sc as plsc

H = 512
W = 512
C = 17
P = H * W
NW = 32
RPW = H // NW
CH = 128
CPR = W // CH
NCH = RPW * CPR
L = 16


def _round(r, cur, wid, iota, hm_hbm, off_hbm, out_hbm, hm_v, off_v, out_v, sems):
    hm_sem, off_sem, out_sem = sems
    row = wid * RPW + (r >> 2)
    w0 = (r & 3) * CH
    base_p = row * W + w0

    pltpu.make_async_copy(hm_hbm.at[0, row, pl.ds(w0, CH)], hm_v[cur], hm_sem[cur]).wait()
    pltpu.make_async_copy(off_hbm.at[0, row, pl.ds(w0, CH)], off_v[cur], off_sem[cur]).wait()

    nxt = 1 - cur
    rn = r + 1

    @pl.when(rn < NCH)
    def _():
        rown = wid * RPW + (rn >> 2)
        w0n = (rn & 3) * CH
        pltpu.make_async_copy(
            hm_hbm.at[0, rown, pl.ds(w0n, CH)], hm_v[nxt], hm_sem[nxt]
        ).start()
        pltpu.make_async_copy(
            off_hbm.at[0, rown, pl.ds(w0n, CH)], off_v[nxt], off_sem[nxt]
        ).start()

    @pl.when(r >= 2)
    def _():
        rp = r - 2
        rowp = wid * RPW + (rp >> 2)
        base_pp = rowp * W + (rp & 3) * CH
        pltpu.make_async_copy(
            out_v[cur], out_hbm.at[0, pl.ds(base_pp, CH)], out_sem[cur]
        ).wait()

    def body(j, _):
        p_loc = j * L + iota
        c0 = jnp.zeros((L,), jnp.int32)
        best = plsc.load_gather(hm_v[cur], [p_loc, c0])
        bestc = c0
        for c in range(1, C):
            v = plsc.load_gather(hm_v[cur], [p_loc, c0 + c])
            gt = v > best
            best = jnp.where(gt, v, best)
            bestc = jnp.where(gt, jnp.full((L,), c, jnp.int32), bestc)
        score = 1.0 / (1.0 + jnp.exp(-best))

        y_off = plsc.load_gather(off_v[cur], [p_loc, bestc])
        x_off = plsc.load_gather(off_v[cur], [p_loc, bestc + C])

        px = (w0 + p_loc).astype(jnp.float32)
        py = jnp.full((L,), 4.0, jnp.float32) * row
        xv = (px * 4.0 + x_off).astype(jnp.int32).astype(jnp.float32)
        yv = (py + y_off).astype(jnp.int32).astype(jnp.float32)

        plsc.store_scatter(out_v[cur], [p_loc, c0], bestc.astype(jnp.float32))
        plsc.store_scatter(out_v[cur], [p_loc, c0 + 1], score)
        plsc.store_scatter(out_v[cur], [p_loc, c0 + 2], xv)
        plsc.store_scatter(out_v[cur], [p_loc, c0 + 3], yv)
        return 0

    lax.fori_loop(0, CH // L, body, 0)

    pltpu.make_async_copy(
        out_v[cur], out_hbm.at[0, pl.ds(base_p, CH)], out_sem[cur]
    ).start()


def _sc_body(
    hm_hbm, off_hbm, out_hbm,
    hm_v0, hm_v1, off_v0, off_v1, out_v0, out_v1,
    hm_s0, hm_s1, off_s0, off_s1, out_s0, out_s1,
):
    wid = lax.axis_index("s") * 2 + lax.axis_index("c")
    iota = lax.iota(jnp.int32, L)
    hm_v = (hm_v0, hm_v1)
    off_v = (off_v0, off_v1)
    out_v = (out_v0, out_v1)
    sems = ((hm_s0, hm_s1), (off_s0, off_s1), (out_s0, out_s1))

    row0 = wid * RPW
    pltpu.make_async_copy(hm_hbm.at[0, row0, pl.ds(0, CH)], hm_v[0], hm_s0).start()
    pltpu.make_async_copy(off_hbm.at[0, row0, pl.ds(0, CH)], off_v[0], off_s0).start()

    def outer(step, _):
        r = step * 2
        _round(r, 0, wid, iota, hm_hbm, off_hbm, out_hbm, hm_v, off_v, out_v, sems)
        _round(r + 1, 1, wid, iota, hm_hbm, off_hbm, out_hbm, hm_v, off_v, out_v, sems)
        return 0

    lax.fori_loop(0, NCH // 2, outer, 0)

    for rp in (NCH - 2, NCH - 1):
        cur = rp % 2
        rowp = wid * RPW + (rp >> 2)
        base_pp = rowp * W + (rp & 3) * CH
        pltpu.make_async_copy(
            out_v[cur], out_hbm.at[0, pl.ds(base_pp, CH)],
            (hm_s0, hm_s1, off_s0, off_s1, out_s0, out_s1)[4 + cur],
        ).wait()


def kernel(heatmaps_input, offsets_input):
    k = functools.partial(
        pl.kernel,
        out_type=jax.ShapeDtypeStruct((1, P, 4), jnp.float32),
        scratch_types=[
            pltpu.VMEM((CH, C), jnp.float32),
            pltpu.VMEM((CH, C), jnp.float32),
            pltpu.VMEM((CH, 2 * C), jnp.float32),
            pltpu.VMEM((CH, 2 * C), jnp.float32),
            pltpu.VMEM((CH, 4), jnp.float32),
            pltpu.VMEM((CH, 4), jnp.float32),
            pltpu.SemaphoreType.DMA,
            pltpu.SemaphoreType.DMA,
            pltpu.SemaphoreType.DMA,
            pltpu.SemaphoreType.DMA,
            pltpu.SemaphoreType.DMA,
            pltpu.SemaphoreType.DMA,
        ],
        mesh=plsc.VectorSubcoreMesh(core_axis_name="c", subcore_axis_name="s"),
        compiler_params=pltpu.CompilerParams(
            use_tc_tiling_on_sc=True, needs_layout_passes=False
        ),
    )(_sc_body)
    return k(heatmaps_input, offsets_input)


# speedup vs baseline: 1.3034x; 1.0124x over previous
import functools
import jax
import jax.numpy as jnp
from jax import lax
from jax.experimental import pallas as pl
from jax.experimental.pallas import tpu as pltpu
from jax.experimental.pallas import tpu_sc as plsc

H = 512
W = 512
C = 17
P = H * W
NW = 32
RPW = H // NW
CH = 128
NCH = RPW * (W // CH)
NB = 3
L = 16


def _chunk_slices(wid, r):
    row = wid * RPW + (r >> 2)
    w0 = (r & 3) * CH
    return row, w0, row * W + w0


def _start_inputs(wid, r, b, hm_hbm, off_hbm, hm_v, off_v, hm_s, off_s):
    row, w0, _ = _chunk_slices(wid, r)
    pltpu.make_async_copy(hm_hbm.at[0, row, pl.ds(w0, CH)], hm_v[b], hm_s[b]).start()
    pltpu.make_async_copy(off_hbm.at[0, row, pl.ds(w0, CH)], off_v[b], off_s[b]).start()


def _round(r, b, wid, iota, hm_hbm, off_hbm, out_hbm,
           hm_v, off_v, out_v, hm_s, off_s, out_s):
    row, w0, base_p = _chunk_slices(wid, r)

    pltpu.make_async_copy(hm_hbm.at[0, row, pl.ds(w0, CH)], hm_v[b], hm_s[b]).wait()
    pltpu.make_async_copy(off_hbm.at[0, row, pl.ds(w0, CH)], off_v[b], off_s[b]).wait()

    rn = r + NB - 1

    @pl.when(rn < NCH)
    def _():
        _start_inputs(
            wid, rn, (b + NB - 1) % NB, hm_hbm, off_hbm, hm_v, off_v, hm_s, off_s
        )

    @pl.when(r >= 1)
    def _():
        _, _, base_pp = _chunk_slices(wid, r - 1)
        pltpu.make_async_copy(
            out_v, out_hbm.at[0, pl.ds(base_pp, CH)], out_s
        ).wait()

    def body(j, _):
        p_loc = j * L + iota
        c0 = jnp.zeros((L,), jnp.int32)
        best = plsc.load_gather(hm_v[b], [p_loc, c0])
        bestc = c0
        for c in range(1, C):
            v = plsc.load_gather(hm_v[b], [p_loc, c0 + c])
            gt = v > best
            best = jnp.where(gt, v, best)
            bestc = jnp.where(gt, jnp.full((L,), c, jnp.int32), bestc)
        score = 1.0 / (1.0 + jnp.exp(-best))

        y_off = plsc.load_gather(off_v[b], [p_loc, bestc])
        x_off = plsc.load_gather(off_v[b], [p_loc, bestc + C])

        px = (w0 + p_loc).astype(jnp.float32)
        py = jnp.full((L,), 4.0, jnp.float32) * row
        xv = (px * 4.0 + x_off).astype(jnp.int32).astype(jnp.float32)
        yv = (py + y_off).astype(jnp.int32).astype(jnp.float32)

        plsc.store_scatter(out_v, [p_loc, c0], bestc.astype(jnp.float32))
        plsc.store_scatter(out_v, [p_loc, c0 + 1], score)
        plsc.store_scatter(out_v, [p_loc, c0 + 2], xv)
        plsc.store_scatter(out_v, [p_loc, c0 + 3], yv)
        return 0

    lax.fori_loop(0, CH // L, body, 0)

    pltpu.make_async_copy(out_v, out_hbm.at[0, pl.ds(base_p, CH)], out_s).start()


def _sc_body(
    hm_hbm, off_hbm, out_hbm,
    hm_v0, hm_v1, hm_v2, off_v0, off_v1, off_v2, out_v,
    hm_s0, hm_s1, hm_s2, off_s0, off_s1, off_s2, out_s,
):
    wid = lax.axis_index("s") * 2 + lax.axis_index("c")
    iota = lax.iota(jnp.int32, L)
    hm_v = (hm_v0, hm_v1, hm_v2)
    off_v = (off_v0, off_v1, off_v2)
    hm_s = (hm_s0, hm_s1, hm_s2)
    off_s = (off_s0, off_s1, off_s2)

    for r0 in range(NB - 1):
        _start_inputs(wid, r0, r0 % NB, hm_hbm, off_hbm, hm_v, off_v, hm_s, off_s)

    def outer(step, _):
        for b in range(NB):
            r = step * NB + b
            _round(r, b, wid, iota, hm_hbm, off_hbm, out_hbm,
                   hm_v, off_v, out_v, hm_s, off_s, out_s)
        return 0

    lax.fori_loop(0, NCH // NB, outer, 0)

    r_last = (NCH // NB) * NB
    _round(r_last, r_last % NB, wid, iota, hm_hbm, off_hbm, out_hbm,
           hm_v, off_v, out_v, hm_s, off_s, out_s)
    _, _, base_pp = _chunk_slices(wid, NCH - 1)
    pltpu.make_async_copy(out_v, out_hbm.at[0, pl.ds(base_pp, CH)], out_s).wait()


def kernel(heatmaps_input, offsets_input):
    k = functools.partial(
        pl.kernel,
        out_type=jax.ShapeDtypeStruct((1, P, 4), jnp.float32),
        scratch_types=[
            pltpu.VMEM((CH, C), jnp.float32),
            pltpu.VMEM((CH, C), jnp.float32),
            pltpu.VMEM((CH, C), jnp.float32),
            pltpu.VMEM((CH, 2 * C), jnp.float32),
            pltpu.VMEM((CH, 2 * C), jnp.float32),
            pltpu.VMEM((CH, 2 * C), jnp.float32),
            pltpu.VMEM((CH, 4), jnp.float32),
            pltpu.SemaphoreType.DMA,
            pltpu.SemaphoreType.DMA,
            pltpu.SemaphoreType.DMA,
            pltpu.SemaphoreType.DMA,
            pltpu.SemaphoreType.DMA,
            pltpu.SemaphoreType.DMA,
            pltpu.SemaphoreType.DMA,
        ],
        mesh=plsc.VectorSubcoreMesh(core_axis_name="c", subcore_axis_name="s"),
        compiler_params=pltpu.CompilerParams(
            use_tc_tiling_on_sc=True, needs_layout_passes=False
        ),
    )(_sc_body)
    return k(heatmaps_input, offsets_input)
